# Initial kernel scaffold; baseline (speedup 1.0000x reference)
import jax
import jax.numpy as jnp
from jax import lax
from jax.experimental import pallas as pl
from jax.experimental.pallas import tpu as pltpu
from jax.experimental.pallas import tpu_sc as plsc

BATCH = 4096
HIST = 200
FEATURE_DIM = 100000
EMBED = 128
NUM_WORKERS = 32
BPW = BATCH // NUM_WORKERS
ZROW = FEATURE_DIM + 1
XSTRIDE = 208

_OFFS = tuple(range(0, 192, 16)) + (184,)


def _sc_body(x_hbm, neg_hbm, tab_hbm, q_out, ub_out, bl_out, nub_out,
             xbuf, nbuf, ebuf0, ebuf1, qidx, cnts, sem):
    wid = lax.axis_index("s") * 2 + lax.axis_index("c")
    base = wid * BPW
    lanes = lax.iota(jnp.int32, 16)

    pltpu.sync_copy(x_hbm.at[pl.ds(base, BPW), :],
                    xbuf.at[:, pl.ds(0, HIST + 1)])
    pltpu.sync_copy(neg_hbm.at[pl.ds(base, BPW), :], nbuf)

    for g in range(BPW // 16):
        rows = lanes + (g * 16)

        def cstep(j, acc, rows=rows):
            cols = jnp.zeros((16,), jnp.int32) + j
            v = plsc.load_gather(xbuf, [rows, cols])
            return acc + (v > 0).astype(jnp.int32)

        cnts[pl.ds(g * 16, 16)] = lax.fori_loop(
            0, HIST, cstep, jnp.zeros((16,), jnp.int32))
    pltpu.sync_copy(cnts, bl_out.at[pl.ds(base, BPW)])

    for g in range(BPW // 16):
        rows = lanes + (g * 16)
        cols = jnp.zeros((16,), jnp.int32) + HIST
        qidx[pl.ds(g * 16, 16)] = plsc.load_gather(xbuf, [rows, cols])

    def remap_x(b, carry):
        for off in _OFFS:
            v = xbuf[b, pl.ds(off, 16)]
            xbuf[b, pl.ds(off, 16)] = jnp.where(v > 0, v, ZROW)
        return carry

    lax.fori_loop(0, BPW, remap_x, 0)

    def remap_n(b, carry):
        for off in _OFFS:
            v = nbuf[b, pl.ds(off, 16)]
            nbuf[b, pl.ds(off, 16)] = jnp.where(v > 0, v, ZROW)
        return carry

    lax.fori_loop(0, BPW, remap_n, 0)

    pltpu.async_copy(tab_hbm.at[qidx], ebuf0.at[pl.ds(0, BPW), :], sem).wait()
    pltpu.sync_copy(ebuf0.at[pl.ds(0, BPW), :], q_out.at[pl.ds(base, BPW)])

    def emit(idx_ref, out_ref):
        def pair(i, carry):
            for k, buf in ((0, ebuf0), (1, ebuf1)):
                b = i * 2 + k
                c1 = pltpu.async_copy(tab_hbm.at[idx_ref.at[b, pl.ds(0, 104)]],
                                      buf.at[pl.ds(0, 104), :], sem)
                c2 = pltpu.async_copy(tab_hbm.at[idx_ref.at[b, pl.ds(104, 96)]],
                                      buf.at[pl.ds(104, 96), :], sem)
                c1.wait()
                c2.wait()
                pltpu.sync_copy(buf, out_ref.at[base + b])
            return carry

        lax.fori_loop(0, BPW // 2, pair, 0)

    emit(xbuf, ub_out)
    emit(nbuf, nub_out)


@jax.jit
def _impl(x, neg_x, table):
    tab2 = jnp.concatenate(
        [table, jnp.zeros((1, EMBED), jnp.float32)], axis=0)
    fn = pl.kernel(
        _sc_body,
        out_type=(
            jax.ShapeDtypeStruct((BATCH, EMBED), jnp.float32),
            jax.ShapeDtypeStruct((BATCH, HIST, EMBED), jnp.float32),
            jax.ShapeDtypeStruct((BATCH,), jnp.int32),
            jax.ShapeDtypeStruct((BATCH, HIST, EMBED), jnp.float32),
        ),
        mesh=plsc.VectorSubcoreMesh(core_axis_name="c", subcore_axis_name="s"),
        scratch_types=[
            pltpu.VMEM((BPW, XSTRIDE), jnp.int32),
            pltpu.VMEM((BPW, HIST), jnp.int32),
            pltpu.VMEM((HIST, EMBED), jnp.float32),
            pltpu.VMEM((HIST, EMBED), jnp.float32),
            pltpu.VMEM((BPW,), jnp.int32),
            pltpu.VMEM((BPW,), jnp.int32),
            pltpu.SemaphoreType.DMA,
        ],
    )
    q, ub, bl, nub = fn(x, neg_x, tab2)
    return q.reshape(BATCH, 1, EMBED), ub, bl, nub


def kernel(x, neg_x, table):
    return _impl(x, neg_x, table)


# speedup vs baseline: 7.3658x; 7.3658x over previous
import jax
import jax.numpy as jnp
from jax import lax
from jax.experimental import pallas as pl
from jax.experimental.pallas import tpu as pltpu
from jax.experimental.pallas import tpu_sc as plsc

BATCH = 4096
HIST = 200
FEATURE_DIM = 100000
EMBED = 128
NUM_WORKERS = 32
BPW = BATCH // NUM_WORKERS
ZROW = FEATURE_DIM + 1
XSTRIDE = 208

_OFFS = tuple(range(0, 192, 16)) + (184,)


def _sc_body(x_hbm, xt_hbm, neg_hbm, tab_hbm, q_out, ub_out, bl_out, nub_out,
             xbuf, xtbuf, nbuf, ebuf0, ebuf1, qidx, cnts, sem):
    wid = lax.axis_index("s") * 2 + lax.axis_index("c")
    base = wid * BPW

    pltpu.sync_copy(x_hbm.at[pl.ds(base * XSTRIDE, BPW * XSTRIDE)], xbuf)
    pltpu.sync_copy(xt_hbm.at[pl.ds(base * XSTRIDE, BPW * XSTRIDE)], xtbuf)
    pltpu.sync_copy(neg_hbm.at[pl.ds(base * HIST, BPW * HIST)], nbuf)

    zeros16 = jnp.zeros((16,), jnp.int32)
    ones16 = jnp.full((16,), 1, jnp.int32)
    zrow16 = jnp.full((16,), ZROW, jnp.int32)

    for c in range(BPW // 16):
        cnts[pl.ds(c * 16, 16)] = zeros16

    def cstep(j, carry):
        for c in range(BPW // 16):
            v = xtbuf[pl.ds(j * BPW + c * 16, 16)]
            cnts[pl.ds(c * 16, 16)] = (
                cnts[pl.ds(c * 16, 16)] + jnp.where(v > zeros16, ones16,
                                                    zeros16))
        return carry

    lax.fori_loop(0, HIST, cstep, 0)
    pltpu.sync_copy(cnts, bl_out.at[pl.ds(base, BPW)])

    for c in range(BPW // 16):
        qidx[pl.ds(c * 16, 16)] = xtbuf[pl.ds(HIST * BPW + c * 16, 16)]

    def remap_x(b, carry):
        for off in _OFFS:
            v = xbuf[pl.ds(b * XSTRIDE + off, 16)]
            xbuf[pl.ds(b * XSTRIDE + off, 16)] = jnp.where(v > zeros16, v,
                                                           zrow16)
        return carry

    lax.fori_loop(0, BPW, remap_x, 0)

    def remap_n(b, carry):
        for off in _OFFS:
            v = nbuf[pl.ds(b * HIST + off, 16)]
            nbuf[pl.ds(b * HIST + off, 16)] = jnp.where(v > zeros16, v, zrow16)
        return carry

    lax.fori_loop(0, BPW, remap_n, 0)

    pltpu.async_copy(tab_hbm.at[qidx], ebuf0.at[pl.ds(0, BPW), :], sem).wait()
    pltpu.sync_copy(ebuf0.at[pl.ds(0, BPW), :], q_out.at[pl.ds(base, BPW)])

    def emit(idx_ref, stride, out_ref):
        def pair(i, carry):
            for k, buf in ((0, ebuf0), (1, ebuf1)):
                b = i * 2 + k
                r = b * stride
                c1 = pltpu.async_copy(tab_hbm.at[idx_ref.at[pl.ds(r, 104)]],
                                      buf.at[pl.ds(0, 104), :], sem)
                c2 = pltpu.async_copy(tab_hbm.at[idx_ref.at[pl.ds(r + 104, 96)]],
                                      buf.at[pl.ds(104, 96), :], sem)
                c1.wait()
                c2.wait()
                pltpu.sync_copy(buf, out_ref.at[base + b])
            return carry

        lax.fori_loop(0, BPW // 2, pair, 0)

    emit(xbuf, XSTRIDE, ub_out)
    emit(nbuf, HIST, nub_out)


@jax.jit
def _impl(x, neg_x, table):
    tab2 = jnp.concatenate(
        [table, jnp.zeros((1, EMBED), jnp.float32)], axis=0)
    xpad = jnp.pad(x, ((0, 0), (0, XSTRIDE - (HIST + 1))))
    xflat = xpad.reshape(-1)
    xtr = (xpad.T.reshape(XSTRIDE, NUM_WORKERS, BPW)
           .transpose(1, 0, 2).reshape(-1))
    negf = neg_x.reshape(-1)
    fn = pl.kernel(
        _sc_body,
        out_type=(
            jax.ShapeDtypeStruct((BATCH, EMBED), jnp.float32),
            jax.ShapeDtypeStruct((BATCH, HIST, EMBED), jnp.float32),
            jax.ShapeDtypeStruct((BATCH,), jnp.int32),
            jax.ShapeDtypeStruct((BATCH, HIST, EMBED), jnp.float32),
        ),
        mesh=plsc.VectorSubcoreMesh(core_axis_name="c", subcore_axis_name="s"),
        scratch_types=[
            pltpu.VMEM((BPW * XSTRIDE,), jnp.int32),
            pltpu.VMEM((BPW * XSTRIDE,), jnp.int32),
            pltpu.VMEM((BPW * HIST,), jnp.int32),
            pltpu.VMEM((HIST, EMBED), jnp.float32),
            pltpu.VMEM((HIST, EMBED), jnp.float32),
            pltpu.VMEM((BPW,), jnp.int32),
            pltpu.VMEM((BPW,), jnp.int32),
            pltpu.SemaphoreType.DMA,
        ],
    )
    q, ub, bl, nub = fn(xflat, xtr, negf, tab2)
    return q.reshape(BATCH, 1, EMBED), ub, bl, nub


def kernel(x, neg_x, table):
    return _impl(x, neg_x, table)


# speedup vs baseline: 9.2421x; 1.2547x over previous
import jax
import jax.numpy as jnp
from jax import lax
from jax.experimental import pallas as pl
from jax.experimental.pallas import tpu as pltpu
from jax.experimental.pallas import tpu_sc as plsc

BATCH = 4096
HIST = 200
FEATURE_DIM = 100000
EMBED = 128
NUM_WORKERS = 32
BPW = BATCH // NUM_WORKERS
ZROW = FEATURE_DIM + 1
XSTRIDE = 208

_OFFS = tuple(range(0, 192, 16)) + (184,)


def _sc_body(x_hbm, xt_hbm, neg_hbm, tab_hbm, q_out, ub_out, bl_out, nub_out,
             xbuf, xtbuf, nbuf, ebuf0, ebuf1, qidx, cnts, sem, sem0, sem1):
    wid = lax.axis_index("s") * 2 + lax.axis_index("c")
    base = wid * BPW

    pltpu.sync_copy(x_hbm.at[pl.ds(base * XSTRIDE, BPW * XSTRIDE)], xbuf)
    pltpu.sync_copy(xt_hbm.at[pl.ds(base * XSTRIDE, BPW * XSTRIDE)], xtbuf)
    pltpu.sync_copy(neg_hbm.at[pl.ds(base * HIST, BPW * HIST)], nbuf)

    zeros16 = jnp.zeros((16,), jnp.int32)
    ones16 = jnp.full((16,), 1, jnp.int32)
    zrow16 = jnp.full((16,), ZROW, jnp.int32)

    for c in range(BPW // 16):
        cnts[pl.ds(c * 16, 16)] = zeros16

    def cstep(j, carry):
        for c in range(BPW // 16):
            v = xtbuf[pl.ds(j * BPW + c * 16, 16)]
            cnts[pl.ds(c * 16, 16)] = (
                cnts[pl.ds(c * 16, 16)] + jnp.where(v > zeros16, ones16,
                                                    zeros16))
        return carry

    lax.fori_loop(0, HIST, cstep, 0)
    pltpu.sync_copy(cnts, bl_out.at[pl.ds(base, BPW)])

    for c in range(BPW // 16):
        qidx[pl.ds(c * 16, 16)] = xtbuf[pl.ds(HIST * BPW + c * 16, 16)]

    def remap_x(b, carry):
        for off in _OFFS:
            v = xbuf[pl.ds(b * XSTRIDE + off, 16)]
            xbuf[pl.ds(b * XSTRIDE + off, 16)] = jnp.where(v > zeros16, v,
                                                           zrow16)
        return carry

    lax.fori_loop(0, BPW, remap_x, 0)

    def remap_n(b, carry):
        for off in _OFFS:
            v = nbuf[pl.ds(b * HIST + off, 16)]
            nbuf[pl.ds(b * HIST + off, 16)] = jnp.where(v > zeros16, v, zrow16)
        return carry

    lax.fori_loop(0, BPW, remap_n, 0)

    pltpu.async_copy(tab_hbm.at[qidx], ebuf0.at[pl.ds(0, BPW), :], sem).wait()
    pltpu.sync_copy(ebuf0.at[pl.ds(0, BPW), :], q_out.at[pl.ds(base, BPW)])

    def start_g(idx_ref, r, buf, sem_):
        pltpu.async_copy(tab_hbm.at[idx_ref.at[pl.ds(r, 104)]],
                         buf.at[pl.ds(0, 104), :], sem_)
        pltpu.async_copy(tab_hbm.at[idx_ref.at[pl.ds(r + 104, 96)]],
                         buf.at[pl.ds(104, 96), :], sem_)

    def drain_g(buf, sem_):
        pltpu.make_async_copy(tab_hbm.at[pl.ds(0, HIST), :], buf, sem_).wait()

    def emit(idx_ref, stride, out_ref):
        start_g(idx_ref, 0, ebuf0, sem0)
        start_g(idx_ref, stride, ebuf1, sem1)

        def pair(i, carry):
            b = 2 * i
            drain_g(ebuf0, sem0)
            pltpu.sync_copy(ebuf0, out_ref.at[base + b])

            @pl.when(b + 2 < BPW)
            def _():
                start_g(idx_ref, (b + 2) * stride, ebuf0, sem0)

            drain_g(ebuf1, sem1)
            pltpu.sync_copy(ebuf1, out_ref.at[base + b + 1])

            @pl.when(b + 3 < BPW)
            def _():
                start_g(idx_ref, (b + 3) * stride, ebuf1, sem1)

            return carry

        lax.fori_loop(0, BPW // 2, pair, 0)

    emit(xbuf, XSTRIDE, ub_out)
    emit(nbuf, HIST, nub_out)


@jax.jit
def _impl(x, neg_x, table):
    tab2 = jnp.concatenate(
        [table, jnp.zeros((1, EMBED), jnp.float32)], axis=0)
    xpad = jnp.pad(x, ((0, 0), (0, XSTRIDE - (HIST + 1))))
    xflat = xpad.reshape(-1)
    xtr = (xpad.T.reshape(XSTRIDE, NUM_WORKERS, BPW)
           .transpose(1, 0, 2).reshape(-1))
    negf = neg_x.reshape(-1)
    fn = pl.kernel(
        _sc_body,
        out_type=(
            jax.ShapeDtypeStruct((BATCH, EMBED), jnp.float32),
            jax.ShapeDtypeStruct((BATCH, HIST, EMBED), jnp.float32),
            jax.ShapeDtypeStruct((BATCH,), jnp.int32),
            jax.ShapeDtypeStruct((BATCH, HIST, EMBED), jnp.float32),
        ),
        mesh=plsc.VectorSubcoreMesh(core_axis_name="c", subcore_axis_name="s"),
        scratch_types=[
            pltpu.VMEM((BPW * XSTRIDE,), jnp.int32),
            pltpu.VMEM((BPW * XSTRIDE,), jnp.int32),
            pltpu.VMEM((BPW * HIST,), jnp.int32),
            pltpu.VMEM((HIST, EMBED), jnp.float32),
            pltpu.VMEM((HIST, EMBED), jnp.float32),
            pltpu.VMEM((BPW,), jnp.int32),
            pltpu.VMEM((BPW,), jnp.int32),
            pltpu.SemaphoreType.DMA,
            pltpu.SemaphoreType.DMA,
            pltpu.SemaphoreType.DMA,
        ],
    )
    q, ub, bl, nub = fn(xflat, xtr, negf, tab2)
    return q.reshape(BATCH, 1, EMBED), ub, bl, nub


def kernel(x, neg_x, table):
    return _impl(x, neg_x, table)
